# baseline (speedup 1.0000x reference)
import functools

import jax
import jax.numpy as jnp
from jax import lax
from jax.experimental import pallas as pl
from jax.experimental.pallas import tpu as pltpu
from jax.experimental.pallas import tpu_sc as plsc

BATCH = 4096
SEQ = 200
DIM = 64
B_TOTAL = BATCH * SEQ
NUM_CORES = 2
NUM_SUBCORES = 16
NW = NUM_CORES * NUM_SUBCORES
ROWS_PER_W = B_TOTAL // NW // SEQ
G0 = 128
G1 = SEQ - G0
NROWS = 1000000
NFULL = NROWS // 128
NTAIL = NROWS - NFULL * 128
BLK_PER_W = (NFULL + NW - 1) // NW
TCOLS = 133

_mesh = plsc.VectorSubcoreMesh(core_axis_name="c", subcore_axis_name="s")


@functools.partial(
    pl.kernel,
    mesh=_mesh,
    out_type=jax.ShapeDtypeStruct((NROWS, 2 * DIM), jnp.float32),
    scratch_types=[
        pltpu.VMEM((DIM, 128), jnp.float32),
        pltpu.VMEM((DIM, 128), jnp.float32),
        pltpu.VMEM((128, TCOLS), jnp.float32),
        pltpu.VMEM((128, TCOLS), jnp.float32),
        pltpu.VMEM((8, 16), jnp.int32),
        pltpu.SemaphoreType.DMA,
        pltpu.SemaphoreType.DMA,
        pltpu.SemaphoreType.DMA,
    ],
    compiler_params=pltpu.CompilerParams(use_tc_tiling_on_sc=True,
                                         needs_layout_passes=False),
)
def _prep_kernel(wt_hbm, tail_hbm, out_hbm, src0, src1, tb0, tb1,
                 base_v, gsem, ssem0, ssem1):
    wid = lax.axis_index("s") * NUM_CORES + lax.axis_index("c")

    lanes = lax.iota(jnp.int32, 16)
    for k in range(8):
        base_v[k] = lanes + 16 * k

    srcs = (src0, src1)
    tbs = (tb0, tb1)
    ssems = (ssem0, ssem1)

    def blk(j):
        return j * NW + wid

    def fire_load(j, buf):
        pltpu.async_copy(wt_hbm.at[:, pl.ds(blk(j) * 128, 128)], buf, gsem)

    def wait_load(buf):
        pltpu.make_async_copy(out_hbm.at[pl.ds(0, DIM)], buf, gsem).wait()

    def transpose(src, tb):
        bases = [base_v[k] for k in range(8)]
        zv = base_v[0] * 0

        def rows(i, carry):
            for u in range(4):
                r = 4 * i + u
                cols = zv + r
                for k in range(8):
                    plsc.store_scatter(tb, [bases[k], cols],
                                       src[r, pl.ds(16 * k, 16)])
            return carry

        lax.fori_loop(0, DIM // 4, rows, 0)

    def fire_store(j, tb, sem):
        pltpu.async_copy(tb.at[:, pl.ds(0, 128)],
                         out_hbm.at[pl.ds(blk(j) * 128, 128)], sem)

    def wait_store(tb, sem):
        pltpu.make_async_copy(tb.at[:, pl.ds(0, 128)],
                              out_hbm.at[pl.ds(0, 128)], sem).wait()

    @pl.when(blk(0) < NFULL)
    def _():
        fire_load(0, srcs[0])

    def body(i, carry):
        for b in range(2):
            j = 2 * i + b
            nb = 1 - b

            @pl.when(blk(j + 1) < NFULL)
            def _():
                if b == 0:
                    @pl.when(i >= 1)
                    def _():
                        wait_store(tbs[nb], ssems[nb])
                else:
                    wait_store(tbs[nb], ssems[nb])
                fire_load(j + 1, srcs[nb])

            @pl.when(blk(j) < NFULL)
            def _():
                wait_load(srcs[b])
                transpose(srcs[b], tbs[b])
                fire_store(j, tbs[b], ssems[b])
        return carry

    lax.fori_loop(0, BLK_PER_W // 2 + 1, body, 0)

    wait_store(tbs[0], ssems[0])
    wait_store(tbs[1], ssems[1])

    @pl.when(wid == NW - 1)
    def _():
        pltpu.sync_copy(tail_hbm, src0.at[pl.ds(0, NTAIL)])
        pltpu.sync_copy(src0.at[pl.ds(0, NTAIL)],
                        out_hbm.at[pl.ds(NFULL * 128, NTAIL)])


@functools.partial(
    pl.kernel,
    mesh=_mesh,
    out_type=jax.ShapeDtypeStruct((B_TOTAL, 2 * DIM), jnp.float32),
    scratch_types=[
        pltpu.VMEM((ROWS_PER_W, SEQ), jnp.int32),
        pltpu.VMEM((SEQ, DIM), jnp.float32),
        pltpu.VMEM((SEQ, DIM), jnp.float32),
        pltpu.SemaphoreType.DMA,
        pltpu.SemaphoreType.DMA,
        pltpu.SemaphoreType.DMA,
    ],
    compiler_params=pltpu.CompilerParams(use_tc_tiling_on_sc=False),
)
def _gather_kernel(idx_hbm, table_hbm, out_hbm, idx_v, rows0, rows1,
                   gsem, ssem0, ssem1):
    wid = lax.axis_index("s") * NUM_CORES + lax.axis_index("c")
    pltpu.sync_copy(idx_hbm.at[wid], idx_v)
    base = wid * ROWS_PER_W

    bufs = (rows0, rows1)
    ssems = (ssem0, ssem1)

    def fire_gathers(g, buf):
        pltpu.async_copy(table_hbm.at[idx_v.at[g, pl.ds(0, G0)]],
                         buf.at[pl.ds(0, G0)], gsem)
        pltpu.async_copy(table_hbm.at[idx_v.at[g, pl.ds(G0, G1)]],
                         buf.at[pl.ds(G0, G1)], gsem)

    def wait_gathers(buf):
        pltpu.make_async_copy(out_hbm.at[pl.ds(0, SEQ), pl.ds(0, DIM)],
                              buf, gsem).wait()

    def store(g, buf, sem):
        pltpu.async_copy(
            buf, out_hbm.at[pl.ds((base + g) * SEQ, SEQ), pl.ds(0, DIM)], sem)

    def wait_store(buf, sem):
        pltpu.make_async_copy(buf, out_hbm.at[pl.ds(0, SEQ), pl.ds(0, DIM)],
                              sem).wait()

    fire_gathers(0, bufs[0])

    def body(i, carry):
        for b in range(2):
            g = 2 * i + b
            nb = 1 - b
            if b == 0:
                @pl.when(i >= 1)
                def _():
                    wait_store(bufs[nb], ssems[nb])
                fire_gathers(g + 1, bufs[nb])
            else:
                @pl.when(i < ROWS_PER_W // 2 - 1)
                def _():
                    wait_store(bufs[nb], ssems[nb])
                    fire_gathers(g + 1, bufs[nb])
            wait_gathers(bufs[b])
            store(g, bufs[b], ssems[b])
        return carry

    lax.fori_loop(0, ROWS_PER_W // 2, body, 0)

    wait_store(bufs[0], ssems[0])
    wait_store(bufs[1], ssems[1])


def kernel(inputs, weight):
    tail = jnp.pad(weight[NFULL * 128:], ((0, 0), (0, DIM)))
    wp = _prep_kernel(weight.T, tail)
    w2 = wp.reshape(2 * NROWS, DIM)
    idx2 = (inputs.astype(jnp.int32) * 2).reshape(NW, ROWS_PER_W, SEQ)
    outp = _gather_kernel(idx2, w2)
    return outp[:, :DIM].reshape(BATCH, SEQ, DIM)


# speedup vs baseline: 1.2622x; 1.2622x over previous
import functools

import jax
import jax.numpy as jnp
from jax import lax
from jax.experimental import pallas as pl
from jax.experimental.pallas import tpu as pltpu
from jax.experimental.pallas import tpu_sc as plsc

BATCH = 4096
SEQ = 200
DIM = 64
B_TOTAL = BATCH * SEQ
NUM_CORES = 2
NUM_SUBCORES = 16
NW = NUM_CORES * NUM_SUBCORES
ROWS_PER_W = B_TOTAL // NW // SEQ
G0 = 128
G1 = SEQ - G0
NROWS = 1000000
NFULL = NROWS // 128
NTAIL = NROWS - NFULL * 128
BLK_PER_W = (NFULL + NW - 1) // NW
TCOLS = 133

_mesh = plsc.VectorSubcoreMesh(core_axis_name="c", subcore_axis_name="s")


@functools.partial(
    pl.kernel,
    mesh=_mesh,
    out_type=jax.ShapeDtypeStruct((NROWS, 2 * DIM), jnp.float32),
    scratch_types=[
        pltpu.VMEM((DIM, 128), jnp.float32),
        pltpu.VMEM((DIM, 128), jnp.float32),
        pltpu.VMEM((128, TCOLS), jnp.float32),
        pltpu.VMEM((128, TCOLS), jnp.float32),
        pltpu.VMEM((8, 16), jnp.int32),
        pltpu.SemaphoreType.DMA,
        pltpu.SemaphoreType.DMA,
        pltpu.SemaphoreType.DMA,
    ],
    compiler_params=pltpu.CompilerParams(use_tc_tiling_on_sc=True,
                                         needs_layout_passes=False),
)
def _prep_kernel(wt_hbm, tail_hbm, out_hbm, src0, src1, tb0, tb1,
                 base_v, gsem, ssem0, ssem1):
    wid = lax.axis_index("s") * NUM_CORES + lax.axis_index("c")

    lanes = lax.iota(jnp.int32, 16)
    for k in range(8):
        base_v[k] = lanes + 16 * k

    srcs = (src0, src1)
    tbs = (tb0, tb1)
    ssems = (ssem0, ssem1)

    def blk(j):
        return j * NW + wid

    def fire_load(j, buf):
        pltpu.async_copy(wt_hbm.at[:, pl.ds(blk(j) * 128, 128)], buf, gsem)

    def wait_load(buf):
        pltpu.make_async_copy(out_hbm.at[pl.ds(0, DIM)], buf, gsem).wait()

    def transpose(src, tb):
        bases = [base_v[k] for k in range(8)]
        zv = base_v[0] * 0

        @plsc.parallel_loop(0, DIM, unroll=4)
        def _rows(r):
            cols = zv + r
            for k in range(8):
                plsc.store_scatter(tb, [bases[k], cols],
                                   src[r, pl.ds(16 * k, 16)])

    def fire_store(j, tb, sem):
        pltpu.async_copy(tb.at[:, pl.ds(0, 128)],
                         out_hbm.at[pl.ds(blk(j) * 128, 128)], sem)

    def wait_store(tb, sem):
        pltpu.make_async_copy(tb.at[:, pl.ds(0, 128)],
                              out_hbm.at[pl.ds(0, 128)], sem).wait()

    @pl.when(blk(0) < NFULL)
    def _():
        fire_load(0, srcs[0])

    def body(i, carry):
        for b in range(2):
            j = 2 * i + b
            nb = 1 - b

            @pl.when(blk(j + 1) < NFULL)
            def _():
                if b == 0:
                    @pl.when(i >= 1)
                    def _():
                        wait_store(tbs[nb], ssems[nb])
                else:
                    wait_store(tbs[nb], ssems[nb])
                fire_load(j + 1, srcs[nb])

            @pl.when(blk(j) < NFULL)
            def _():
                wait_load(srcs[b])
                transpose(srcs[b], tbs[b])
                fire_store(j, tbs[b], ssems[b])
        return carry

    lax.fori_loop(0, BLK_PER_W // 2 + 1, body, 0)

    wait_store(tbs[0], ssems[0])
    wait_store(tbs[1], ssems[1])

    @pl.when(wid == NW - 1)
    def _():
        pltpu.sync_copy(tail_hbm, src0.at[pl.ds(0, NTAIL)])
        pltpu.sync_copy(src0.at[pl.ds(0, NTAIL)],
                        out_hbm.at[pl.ds(NFULL * 128, NTAIL)])


@functools.partial(
    pl.kernel,
    mesh=_mesh,
    out_type=jax.ShapeDtypeStruct((B_TOTAL, 2 * DIM), jnp.float32),
    scratch_types=[
        pltpu.VMEM((ROWS_PER_W, SEQ), jnp.int32),
        pltpu.VMEM((SEQ, DIM), jnp.float32),
        pltpu.VMEM((SEQ, DIM), jnp.float32),
        pltpu.SemaphoreType.DMA,
        pltpu.SemaphoreType.DMA,
        pltpu.SemaphoreType.DMA,
    ],
    compiler_params=pltpu.CompilerParams(use_tc_tiling_on_sc=False),
)
def _gather_kernel(idx_hbm, table_hbm, out_hbm, idx_v, rows0, rows1,
                   gsem, ssem0, ssem1):
    wid = lax.axis_index("s") * NUM_CORES + lax.axis_index("c")
    pltpu.sync_copy(idx_hbm.at[wid], idx_v)
    base = wid * ROWS_PER_W

    bufs = (rows0, rows1)
    ssems = (ssem0, ssem1)

    def fire_gathers(g, buf):
        pltpu.async_copy(table_hbm.at[idx_v.at[g, pl.ds(0, G0)]],
                         buf.at[pl.ds(0, G0)], gsem)
        pltpu.async_copy(table_hbm.at[idx_v.at[g, pl.ds(G0, G1)]],
                         buf.at[pl.ds(G0, G1)], gsem)

    def wait_gathers(buf):
        pltpu.make_async_copy(out_hbm.at[pl.ds(0, SEQ), pl.ds(0, DIM)],
                              buf, gsem).wait()

    def store(g, buf, sem):
        pltpu.async_copy(
            buf, out_hbm.at[pl.ds((base + g) * SEQ, SEQ), pl.ds(0, DIM)], sem)

    def wait_store(buf, sem):
        pltpu.make_async_copy(buf, out_hbm.at[pl.ds(0, SEQ), pl.ds(0, DIM)],
                              sem).wait()

    fire_gathers(0, bufs[0])

    def body(i, carry):
        for b in range(2):
            g = 2 * i + b
            nb = 1 - b
            if b == 0:
                @pl.when(i >= 1)
                def _():
                    wait_store(bufs[nb], ssems[nb])
                fire_gathers(g + 1, bufs[nb])
            else:
                @pl.when(i < ROWS_PER_W // 2 - 1)
                def _():
                    wait_store(bufs[nb], ssems[nb])
                    fire_gathers(g + 1, bufs[nb])
            wait_gathers(bufs[b])
            store(g, bufs[b], ssems[b])
        return carry

    lax.fori_loop(0, ROWS_PER_W // 2, body, 0)

    wait_store(bufs[0], ssems[0])
    wait_store(bufs[1], ssems[1])


def kernel(inputs, weight):
    tail = jnp.pad(weight[NFULL * 128:], ((0, 0), (0, DIM)))
    wp = _prep_kernel(weight.T, tail)
    w2 = wp.reshape(2 * NROWS, DIM)
    idx2 = (inputs.astype(jnp.int32) * 2).reshape(NW, ROWS_PER_W, SEQ)
    outp = _gather_kernel(idx2, w2)
    return outp[:, :DIM].reshape(BATCH, SEQ, DIM)


# speedup vs baseline: 1.9722x; 1.5626x over previous
import functools

import jax
import jax.numpy as jnp
from jax import lax
from jax.experimental import pallas as pl
from jax.experimental.pallas import tpu as pltpu
from jax.experimental.pallas import tpu_sc as plsc

BATCH = 4096
SEQ = 200
DIM = 64
B_TOTAL = BATCH * SEQ
NUM_CORES = 2
NUM_SUBCORES = 16
NW = NUM_CORES * NUM_SUBCORES
ROWS_PER_W = B_TOTAL // NW // SEQ
G0 = 128
G1 = SEQ - G0

_mesh = plsc.VectorSubcoreMesh(core_axis_name="c", subcore_axis_name="s")


@functools.partial(
    pl.kernel,
    mesh=_mesh,
    out_type=jax.ShapeDtypeStruct((B_TOTAL, 2 * DIM), jnp.float32),
    scratch_types=[
        pltpu.VMEM((ROWS_PER_W, SEQ), jnp.int32),
        pltpu.VMEM((SEQ, DIM), jnp.float32),
        pltpu.VMEM((SEQ, DIM), jnp.float32),
        pltpu.SemaphoreType.DMA,
        pltpu.SemaphoreType.DMA,
        pltpu.SemaphoreType.DMA,
    ],
    compiler_params=pltpu.CompilerParams(use_tc_tiling_on_sc=False),
)
def _gather_kernel(idx_hbm, table_hbm, out_hbm, idx_v, rows0, rows1,
                   gsem, ssem0, ssem1):
    wid = lax.axis_index("s") * NUM_CORES + lax.axis_index("c")
    pltpu.sync_copy(idx_hbm.at[wid], idx_v)
    base = wid * ROWS_PER_W

    bufs = (rows0, rows1)
    ssems = (ssem0, ssem1)

    def fire_gathers(g, buf):
        pltpu.async_copy(table_hbm.at[idx_v.at[g, pl.ds(0, G0)]],
                         buf.at[pl.ds(0, G0)], gsem)
        pltpu.async_copy(table_hbm.at[idx_v.at[g, pl.ds(G0, G1)]],
                         buf.at[pl.ds(G0, G1)], gsem)

    def wait_gathers(buf):
        pltpu.make_async_copy(out_hbm.at[pl.ds(0, SEQ), pl.ds(0, DIM)],
                              buf, gsem).wait()

    def store(g, buf, sem):
        pltpu.async_copy(
            buf, out_hbm.at[pl.ds((base + g) * SEQ, SEQ), pl.ds(0, DIM)], sem)

    def wait_store(buf, sem):
        pltpu.make_async_copy(buf, out_hbm.at[pl.ds(0, SEQ), pl.ds(0, DIM)],
                              sem).wait()

    fire_gathers(0, bufs[0])

    def body(i, carry):
        for b in range(2):
            g = 2 * i + b
            nb = 1 - b
            if b == 0:
                @pl.when(i >= 1)
                def _():
                    wait_store(bufs[nb], ssems[nb])
                fire_gathers(g + 1, bufs[nb])
            else:
                @pl.when(i < ROWS_PER_W // 2 - 1)
                def _():
                    wait_store(bufs[nb], ssems[nb])
                    fire_gathers(g + 1, bufs[nb])
            wait_gathers(bufs[b])
            store(g, bufs[b], ssems[b])
        return carry

    lax.fori_loop(0, ROWS_PER_W // 2, body, 0)

    wait_store(bufs[0], ssems[0])
    wait_store(bufs[1], ssems[1])


def kernel(inputs, weight):
    w2 = jnp.pad(weight, ((0, 0), (0, DIM))).reshape(2 * weight.shape[0], DIM)
    idx2 = (inputs.astype(jnp.int32) * 2).reshape(NW, ROWS_PER_W, SEQ)
    outp = _gather_kernel(idx2, w2)
    return outp[:, :DIM].reshape(BATCH, SEQ, DIM)
